# baseline (speedup 1.0000x reference)
import functools

import jax
import jax.numpy as jnp
from jax import lax
from jax.experimental import pallas as pl
from jax.experimental.pallas import tpu as pltpu
from jax.experimental.pallas import tpu_sc as plsc

N_NODES = 10000
N_EDGES = 160000
HIDDEN = 256
NCHUNK = 4
CW = HIDDEN // NCHUNK
NCORES = 2
NSUB = 16
EDGES_PER_TILE = N_EDGES // NSUB
BLK = 48
FROWS = BLK * CW // 128
NBLKF = 208
NBLKT = NBLKF + 1
TROWS = 8
TPAD = NBLKT * BLK
N_ACC = 10240
STRIPE = N_ACC // NSUB
ZROWS = 8
PROWS = N_EDGES // 2


def _mm_body(x_ref, w_ref, b_ref, o_ref):
    o_ref[...] = (
        jnp.dot(x_ref[...], w_ref[...], preferred_element_type=jnp.float32)
        + b_ref[...]
    )


def _matmul_bias(x, w, b, bm):
    m, k = x.shape
    n = w.shape[1]
    return pl.pallas_call(
        _mm_body,
        grid=(pl.cdiv(m, bm),),
        in_specs=[
            pl.BlockSpec((bm, k), lambda i: (i, 0)),
            pl.BlockSpec((k, n), lambda i: (0, 0)),
            pl.BlockSpec((1, n), lambda i: (0, 0)),
        ],
        out_specs=pl.BlockSpec((bm, n), lambda i: (i, 0)),
        out_shape=jax.ShapeDtypeStruct((m, n), jnp.float32),
    )(x, w, b.reshape(1, n))


def _pack2_weights(w, b):
    k = w.shape[0]
    w4 = w.reshape(k, NCHUNK, CW)
    w2 = jnp.zeros((2, k, NCHUNK, 2, CW), jnp.float32)
    w2 = w2.at[0, :, :, 0, :].set(w4)
    w2 = w2.at[1, :, :, 1, :].set(w4)
    w2 = w2.reshape(2 * k, NCHUNK * 128)
    b2 = jnp.tile(b.reshape(NCHUNK, 1, CW), (1, 2, 1)).reshape(NCHUNK * 128)
    return w2, b2


def _pack2_weights_ce(w, b):
    c4 = w.reshape(NCHUNK, CW, NCHUNK, CW)
    w2 = jnp.zeros((NCHUNK, 2, CW, NCHUNK, 2, CW), jnp.float32)
    w2 = w2.at[:, 0, :, :, 0, :].set(c4)
    w2 = w2.at[:, 1, :, :, 1, :].set(c4)
    w2 = w2.reshape(NCHUNK * 128, NCHUNK * 128)
    b2 = jnp.tile(b.reshape(NCHUNK, 1, CW), (1, 2, 1)).reshape(NCHUNK * 128)
    return w2, b2


_EROWS = N_EDGES // 2
_SBLK = 2000
_SNB = _EROWS // _SBLK


def _stats_body(x_ref, s_ref, q_ref):
    i = pl.program_id(0)

    @pl.when(i == 0)
    def _():
        s_ref[...] = jnp.zeros_like(s_ref)
        q_ref[...] = jnp.zeros_like(q_ref)

    x = x_ref[...]
    s_ref[...] += jnp.sum(x, axis=0, keepdims=True)
    q_ref[...] += jnp.sum(x * x, axis=0, keepdims=True)


def _bn_stats(enewf):
    return pl.pallas_call(
        _stats_body,
        grid=(_SNB,),
        in_specs=[pl.BlockSpec((_SBLK, 512), lambda i: (i, 0))],
        out_specs=[
            pl.BlockSpec((1, 512), lambda i: (0, 0)),
            pl.BlockSpec((1, 512), lambda i: (0, 0)),
        ],
        out_shape=[jax.ShapeDtypeStruct((1, 512), jnp.float32)] * 2,
    )(enewf)


def _eupd_body(en_ref, ec_ref, sc_ref, sh_ref, o_ref, es_ref):
    i = pl.program_id(0)

    @pl.when(i == 0)
    def _():
        es_ref[...] = jnp.zeros_like(es_ref)

    out = ec_ref[...] + jnp.maximum(
        en_ref[...] * sc_ref[...] + sh_ref[...], 0.0
    )
    o_ref[...] = out
    es_ref[...] += jnp.sum(out, axis=0, keepdims=True)


def _e_update(enewf, ecf, scale, shift):
    return pl.pallas_call(
        _eupd_body,
        grid=(_SNB,),
        in_specs=[
            pl.BlockSpec((_SBLK, 512), lambda i: (i, 0)),
            pl.BlockSpec((_SBLK, 512), lambda i: (i, 0)),
            pl.BlockSpec((1, 512), lambda i: (0, 0)),
            pl.BlockSpec((1, 512), lambda i: (0, 0)),
        ],
        out_specs=[
            pl.BlockSpec((_SBLK, 512), lambda i: (i, 0)),
            pl.BlockSpec((1, 512), lambda i: (0, 0)),
        ],
        out_shape=[
            jax.ShapeDtypeStruct((_EROWS, 512), jnp.float32),
            jax.ShapeDtypeStruct((1, 512), jnp.float32),
        ],
    )(enewf, ecf, scale, shift)


def _ceupd_body(en_ref, ec_ref, sc_ref, sh_ref, w_ref, b_ref, oe_ref, oc_ref):
    out = ec_ref[...] + jnp.maximum(
        en_ref[...] * sc_ref[...] + sh_ref[...], 0.0
    )
    oe_ref[...] = out
    oc_ref[...] = (
        jnp.dot(out, w_ref[...], preferred_element_type=jnp.float32)
        + b_ref[...]
    )


def _ce_fused(enewf, ecf, scale, shift, w2, b2):
    return pl.pallas_call(
        _ceupd_body,
        grid=(_SNB,),
        in_specs=[
            pl.BlockSpec((_SBLK, 512), lambda i: (i, 0)),
            pl.BlockSpec((_SBLK, 512), lambda i: (i, 0)),
            pl.BlockSpec((1, 512), lambda i: (0, 0)),
            pl.BlockSpec((1, 512), lambda i: (0, 0)),
            pl.BlockSpec((512, 512), lambda i: (0, 0)),
            pl.BlockSpec((1, 512), lambda i: (0, 0)),
        ],
        out_specs=[
            pl.BlockSpec((_SBLK, 512), lambda i: (i, 0)),
            pl.BlockSpec((_SBLK, 512), lambda i: (i, 0)),
        ],
        out_shape=[
            jax.ShapeDtypeStruct((_EROWS, 512), jnp.float32),
            jax.ShapeDtypeStruct((_EROWS, 512), jnp.float32),
        ],
    )(enewf, ecf, scale, shift, w2, b2.reshape(1, 512))


def _proj_body(x_ref, w_ref, b_ref, ah_ref, sd_ref, ee_ref):
    res = (
        jnp.dot(x_ref[...], w_ref[...], preferred_element_type=jnp.float32)
        + b_ref[...]
    )
    ah_ref[...] = res[:, :HIDDEN]
    for c in range(NCHUNK):
        sd_ref[c] = jnp.concatenate(
            [
                res[:, 2 * HIDDEN + c * CW : 2 * HIDDEN + (c + 1) * CW],
                res[:, HIDDEN + c * CW : HIDDEN + (c + 1) * CW],
            ],
            axis=1,
        )
    for k2 in range(NCORES):
        ee_ref[k2] = res[:, 3 * HIDDEN + k2 * 128 : 3 * HIDDEN + (k2 + 1) * 128]


def _proj(h, w_cat, b_cat, bm=1000):
    m = h.shape[0]
    return pl.pallas_call(
        _proj_body,
        grid=(pl.cdiv(m, bm),),
        in_specs=[
            pl.BlockSpec((bm, HIDDEN), lambda i: (i, 0)),
            pl.BlockSpec((HIDDEN, 4 * HIDDEN), lambda i: (0, 0)),
            pl.BlockSpec((1, 4 * HIDDEN), lambda i: (0, 0)),
        ],
        out_specs=[
            pl.BlockSpec((bm, HIDDEN), lambda i: (i, 0)),
            pl.BlockSpec((NCHUNK, bm, 128), lambda i: (0, i, 0)),
            pl.BlockSpec((NCORES, bm, 128), lambda i: (0, i, 0)),
        ],
        out_shape=[
            jax.ShapeDtypeStruct((m, HIDDEN), jnp.float32),
            jax.ShapeDtypeStruct((NCHUNK, m, 128), jnp.float32),
            jax.ShapeDtypeStruct((NCORES, m, 128), jnp.float32),
        ],
    )(h, w_cat, b_cat.reshape(1, 4 * HIDDEN))


def _edge_body(
    gadj_hbm, dstp_hbm, ce_hbm, sd_hbm, ee_hbm,
    enew_hbm, acc_hbm,
    zbuf,
    gA, gB, dA, dB, ceA, ceB, sdA, sdB, eeA, eeB, sgA, sgB,
    acc_sh,
    iA, iB, dsA, dsB, gsA, gsB, geA, geB, csA, csB, wA, wB, ssA, ssB,
):
    core = lax.axis_index("c")
    sub = lax.axis_index("s")
    bufs = [
        (gA, dA, ceA, sdA, eeA, sgA, iA, dsA, gsA, geA, csA, wA, ssA),
        (gB, dB, ceB, sdB, eeB, sgB, iB, dsB, gsB, geB, csB, wB, ssB),
    ]

    zero16 = jnp.zeros((16,), jnp.float32)
    for i in range(ZROWS):
        for kk in range(8):
            zbuf[i, pl.ds(kk * 16, 16)] = zero16

    for p in range(NCHUNK // NCORES):
        c = core * (NCHUNK // NCORES) + p

        coff = pl.multiple_of(c * 128, 128)

        def ce_row(j):
            return pl.multiple_of(
                sub * (EDGES_PER_TILE // 2) + j * FROWS, 8
            )

        def fire_idx(j, P):
            g, dv = bufs[P][0], bufs[P][1]
            semI, semD = bufs[P][6], bufs[P][7]
            goff = pl.multiple_of(((c * NSUB + sub) * NBLKT + j) * 128, 8)
            pltpu.async_copy(gadj_hbm.at[pl.ds(goff, 128)], g, semI)
            doff = pl.multiple_of(sub * TPAD + j * BLK, 8)
            pltpu.async_copy(dstp_hbm.at[pl.ds(doff, BLK)], dv, semD)

        def fire_gather(j, P, wait_w, frows):
            g, dv, ceb, sdb, eeb, sgb = (
                bufs[P][0], bufs[P][1], bufs[P][2],
                bufs[P][3], bufs[P][4], bufs[P][5],
            )
            semI, semG, semE, semC, semW, semS = (
                bufs[P][6], bufs[P][8], bufs[P][9],
                bufs[P][10], bufs[P][11], bufs[P][12],
            )
            pltpu.make_async_copy(gadj_hbm.at[pl.ds(0, 128)], g, semI).wait()
            if wait_w:
                pltpu.make_async_copy(
                    ceb, enew_hbm.at[pl.ds(0, FROWS), pl.ds(0, 128)], semW
                ).wait()
                pltpu.make_async_copy(sgb, acc_sh.at[dv], semS).wait()
            pltpu.async_copy(sd_hbm.at[g.at[pl.ds(0, BLK)]], sdb, semG)
            pltpu.async_copy(ee_hbm.at[g.at[pl.ds(BLK, BLK)]], eeb, semE)
            ce_dst = ceb if frows == FROWS else ceb.at[pl.ds(0, frows)]
            pltpu.async_copy(
                ce_hbm.at[pl.ds(ce_row(j), frows), pl.ds(coff, 128)], ce_dst, semC
            )

        def compute_block(j, P, qrows, prefetch=None):
            g, dv, ceb, sdb, eeb, sgb = (
                bufs[P][0], bufs[P][1], bufs[P][2],
                bufs[P][3], bufs[P][4], bufs[P][5],
            )
            semD, semG, semE, semC, semW, semS = (
                bufs[P][7], bufs[P][8], bufs[P][9],
                bufs[P][10], bufs[P][11], bufs[P][12],
            )
            pltpu.make_async_copy(
                sd_hbm.at[g.at[pl.ds(0, BLK)]], sdb, semG
            ).wait()
            ce_dst = ceb if qrows == FROWS else ceb.at[pl.ds(0, qrows)]
            if prefetch is not None:
                prefetch()
            pltpu.make_async_copy(
                ee_hbm.at[g.at[pl.ds(BLK, BLK)]], eeb, semE
            ).wait()
            pltpu.make_async_copy(
                ce_hbm.at[pl.ds(0, qrows), pl.ds(0, 128)], ce_dst, semC
            ).wait()
            pltpu.make_async_copy(dstp_hbm.at[pl.ds(0, BLK)], dv, semD).wait()

            @plsc.parallel_loop(0, qrows, unroll=2)
            def comp(q):
                for half in range(2):
                    r = 2 * q + half
                    for kk in range(CW // 16):
                        csl = pl.ds(half * CW + kk * 16, 16)
                        sl = pl.ds(kk * 16, 16)
                        esl = pl.ds(p * CW + kk * 16, 16)
                        bsl = pl.ds(CW + kk * 16, 16)
                        en = ceb[q, csl] + sdb[r, sl] + eeb[r, esl]
                        ceb[q, csl] = en
                        sg = 1.0 / (1.0 + jnp.exp(-en))
                        sgb[r, bsl] = sg
                        sgb[r, sl] = sg * sdb[r, bsl]
            pltpu.async_copy(
                ce_dst,
                enew_hbm.at[pl.ds(ce_row(j), qrows), pl.ds(coff, 128)],
                semW,
            )
            pltpu.async_copy(sgb, acc_sh.at[dv], semS, add=True)

        for t in range(STRIPE // ZROWS):
            zr0 = pl.multiple_of(sub * STRIPE + t * ZROWS, 8)
            pltpu.sync_copy(zbuf, acc_sh.at[pl.ds(zr0, ZROWS)])
        plsc.subcore_barrier()

        def pair(j, carry):
            def pf_a():
                @pl.when(j + 2 < NBLKF)
                def _():
                    fire_idx(j + 2, 0)

            def pf_b():
                @pl.when(j + 3 < NBLKF)
                def _():
                    fire_idx(j + 3, 1)

            compute_block(j, 0, FROWS, prefetch=pf_a)

            @pl.when(j + 2 < NBLKF)
            def _():
                fire_gather(j + 2, 0, True, FROWS)

            compute_block(j + 1, 1, FROWS, prefetch=pf_b)

            @pl.when(j + 3 < NBLKF)
            def _():
                fire_gather(j + 3, 1, True, FROWS)

            return carry

        fire_idx(0, 0)
        fire_idx(1, 1)
        fire_gather(0, 0, False, FROWS)
        fire_gather(1, 1, False, FROWS)
        lax.fori_loop(0, NBLKF // 2, lambda jj, cc: pair(2 * jj, cc), 0)
        fire_idx(NBLKF, 0)
        fire_gather(NBLKF, 0, True, TROWS)
        compute_block(NBLKF, 0, TROWS)

        pltpu.make_async_copy(
            ceA.at[pl.ds(0, TROWS)],
            enew_hbm.at[pl.ds(0, TROWS), pl.ds(0, 128)],
            wA,
        ).wait()
        pltpu.make_async_copy(
            ceB, enew_hbm.at[pl.ds(0, FROWS), pl.ds(0, 128)], wB
        ).wait()
        pltpu.make_async_copy(sgA, acc_sh.at[dA], ssA).wait()
        pltpu.make_async_copy(sgB, acc_sh.at[dB], ssB).wait()
        plsc.subcore_barrier()

        fr0 = pl.multiple_of(sub * STRIPE, 8)
        fo0 = pl.multiple_of(c * N_ACC + sub * STRIPE, 8)
        pltpu.sync_copy(
            acc_sh.at[pl.ds(fr0, STRIPE)],
            acc_hbm.at[pl.ds(fo0, STRIPE)],
        )


_edge_kernel = functools.partial(
    pl.kernel,
    out_type=[
        jax.ShapeDtypeStruct((PROWS, 512), jnp.float32),
        jax.ShapeDtypeStruct((NCHUNK * N_ACC, 128), jnp.float32),
    ],
    mesh=plsc.VectorSubcoreMesh(core_axis_name="c", subcore_axis_name="s"),
    scratch_types=[
        pltpu.VMEM((ZROWS, 128), jnp.float32),
        pltpu.VMEM((128,), jnp.int32),
        pltpu.VMEM((128,), jnp.int32),
        pltpu.VMEM((BLK,), jnp.int32),
        pltpu.VMEM((BLK,), jnp.int32),
        pltpu.VMEM((FROWS, 128), jnp.float32),
        pltpu.VMEM((FROWS, 128), jnp.float32),
        pltpu.VMEM((BLK, 128), jnp.float32),
        pltpu.VMEM((BLK, 128), jnp.float32),
        pltpu.VMEM((BLK, 128), jnp.float32),
        pltpu.VMEM((BLK, 128), jnp.float32),
        pltpu.VMEM((BLK, 128), jnp.float32),
        pltpu.VMEM((BLK, 128), jnp.float32),
        pltpu.VMEM_SHARED((N_ACC, 128), jnp.float32),
    ] + [pltpu.SemaphoreType.DMA] * 14,
)(_edge_body)


def _bn(x, g, b, axis):
    mu = jnp.mean(x, axis=axis, keepdims=True)
    var = jnp.var(x, axis=axis, keepdims=True)
    return g * (x - mu) / jnp.sqrt(var + 1e-5) + b


def kernel(dataset_idx, edge_index, h, e, training_flag, params):
    src = edge_index[0]
    dst = edge_index[1]
    srcr = src.reshape(NSUB, EDGES_PER_TILE)
    dstr = dst.reshape(NSUB, EDGES_PER_TILE)
    pad = TPAD - EDGES_PER_TILE
    srcb = jnp.pad(srcr, ((0, 0), (0, pad))).reshape(NSUB, NBLKT, BLK)
    dstb = jnp.pad(dstr, ((0, 0), (0, pad))).reshape(NSUB, NBLKT, BLK)
    zpad = jnp.zeros((NSUB, NBLKT, 128 - 2 * BLK), jnp.int32)
    gadj = jnp.stack(
        [
            jnp.concatenate(
                [srcb + c * N_NODES, dstb + (c // 2) * N_NODES, zpad], axis=-1
            )
            for c in range(NCHUNK)
        ]
    ).reshape(-1)
    dstp = jnp.pad(
        dstr, ((0, 0), (0, pad)), constant_values=N_NODES
    ).reshape(-1)
    h = _matmul_bias(h, params["emb_h_W"], params["emb_h_b"], bm=1000)
    w2e, b2e = _pack2_weights(params["emb_e_W"], params["emb_e_b"])
    ecf = _matmul_bias(e.reshape(PROWS, 32), w2e, b2e, bm=2000)

    esum = None
    pend = None
    for lp in params["layers"]:
        w_cat = jnp.concatenate(
            [lp["A_W"], lp["B_W"], lp["D_W"], lp["E_W"]], axis=1
        )
        b_cat = jnp.concatenate(
            [lp["A_b"], lp["B_b"], lp["D_b"], lp["E_b"]], axis=0
        )
        Ah, sdT, eeT = _proj(h, w_cat, b_cat)
        w2c, b2c = _pack2_weights_ce(lp["C_W"], lp["C_b"])
        if pend is None:
            ceF = _matmul_bias(ecf, w2c, b2c, bm=2000)
        else:
            ecf, ceF = _ce_fused(pend[0], ecf, pend[1], pend[2], w2c, b2c)

        enewF, accF = _edge_kernel(
            gadj, dstp,
            ceF,
            sdT.reshape(NCHUNK * N_NODES, 128),
            eeT.reshape(NCORES * N_NODES, 128),
        )
        accR = accF.reshape(NCHUNK, N_ACC, 128)[:, :N_NODES]
        num = accR[:, :, :CW].transpose(1, 0, 2).reshape(N_NODES, HIDDEN)
        den = accR[:, :, CW:].transpose(1, 0, 2).reshape(N_NODES, HIDDEN)

        h_new = Ah + num / (den + 1e-6)
        h_new = jax.nn.relu(_bn(h_new, lp["bn_h_g"], lp["bn_h_b"], axis=0))
        h = h + h_new

        S, Q = _bn_stats(enewF)
        s4 = S.reshape(NCHUNK, 2, CW).sum(1)
        q4 = Q.reshape(NCHUNK, 2, CW).sum(1)
        mu = s4 / N_EDGES
        var = q4 / N_EDGES - mu * mu
        g4 = lp["bn_e_g"].reshape(NCHUNK, CW)
        b4 = lp["bn_e_b"].reshape(NCHUNK, CW)
        scale4 = g4 / jnp.sqrt(var + 1e-5)
        shift4 = b4 - mu * scale4
        scale = jnp.tile(scale4.reshape(NCHUNK, 1, CW), (1, 2, 1)).reshape(1, 512)
        shift = jnp.tile(shift4.reshape(NCHUNK, 1, CW), (1, 2, 1)).reshape(1, 512)
        pend = (enewF, scale, shift)

    ecf, esum = _e_update(pend[0], ecf, pend[1], pend[2])

    e_sum = esum.reshape(NCHUNK, 2, CW).sum(1).reshape(HIDDEN)
    hg = jnp.concatenate([jnp.sum(h, axis=0), e_sum])
    x = hg
    n = len(params["mlp_Ws"])
    for i in range(n):
        x = x @ params["mlp_Ws"][i] + params["mlp_bs"][i]
        if i < n - 1:
            x = jax.nn.relu(x)
    return x


# speedup vs baseline: 1.2146x; 1.2146x over previous
import functools

import jax
import jax.numpy as jnp
from jax import lax
from jax.experimental import pallas as pl
from jax.experimental.pallas import tpu as pltpu
from jax.experimental.pallas import tpu_sc as plsc

N_NODES = 10000
N_EDGES = 160000
HIDDEN = 256
NCHUNK = 4
CW = HIDDEN // NCHUNK
NCORES = 2
NSUB = 16
EDGES_PER_TILE = N_EDGES // NSUB
BLK = 48
FROWS = BLK * CW // 128
NBLKF = 208
NBLKT = NBLKF + 1
TROWS = 8
TPAD = NBLKT * BLK
N_ACC = 10240
STRIPE = N_ACC // NSUB
ZROWS = 8
PROWS = N_EDGES // 2


def _mm_body(x_ref, w_ref, b_ref, o_ref):
    o_ref[...] = (
        jnp.dot(x_ref[...], w_ref[...], preferred_element_type=jnp.float32)
        + b_ref[...]
    )


def _matmul_bias(x, w, b, bm):
    m, k = x.shape
    n = w.shape[1]
    return pl.pallas_call(
        _mm_body,
        grid=(pl.cdiv(m, bm),),
        in_specs=[
            pl.BlockSpec((bm, k), lambda i: (i, 0)),
            pl.BlockSpec((k, n), lambda i: (0, 0)),
            pl.BlockSpec((1, n), lambda i: (0, 0)),
        ],
        out_specs=pl.BlockSpec((bm, n), lambda i: (i, 0)),
        out_shape=jax.ShapeDtypeStruct((m, n), jnp.float32),
    )(x, w, b.reshape(1, n))


def _pack2_weights(w, b):
    k = w.shape[0]
    w4 = w.reshape(k, NCHUNK, CW)
    w2 = jnp.zeros((2, k, NCHUNK, 2, CW), jnp.float32)
    w2 = w2.at[0, :, :, 0, :].set(w4)
    w2 = w2.at[1, :, :, 1, :].set(w4)
    w2 = w2.reshape(2 * k, NCHUNK * 128)
    b2 = jnp.tile(b.reshape(NCHUNK, 1, CW), (1, 2, 1)).reshape(NCHUNK * 128)
    return w2, b2


def _pack2_weights_ce(w, b):
    c4 = w.reshape(NCHUNK, CW, NCHUNK, CW)
    w2 = jnp.zeros((NCHUNK, 2, CW, NCHUNK, 2, CW), jnp.float32)
    w2 = w2.at[:, 0, :, :, 0, :].set(c4)
    w2 = w2.at[:, 1, :, :, 1, :].set(c4)
    w2 = w2.reshape(NCHUNK * 128, NCHUNK * 128)
    b2 = jnp.tile(b.reshape(NCHUNK, 1, CW), (1, 2, 1)).reshape(NCHUNK * 128)
    return w2, b2


_EROWS = N_EDGES // 2
_SBLK = 2000
_SNB = _EROWS // _SBLK


def _stats_body(x_ref, s_ref, q_ref):
    i = pl.program_id(0)

    @pl.when(i == 0)
    def _():
        s_ref[...] = jnp.zeros_like(s_ref)
        q_ref[...] = jnp.zeros_like(q_ref)

    x = x_ref[...]
    s_ref[...] += jnp.sum(x, axis=0, keepdims=True)
    q_ref[...] += jnp.sum(x * x, axis=0, keepdims=True)


def _bn_stats(enewf):
    return pl.pallas_call(
        _stats_body,
        grid=(_SNB,),
        in_specs=[pl.BlockSpec((_SBLK, 512), lambda i: (i, 0))],
        out_specs=[
            pl.BlockSpec((1, 512), lambda i: (0, 0)),
            pl.BlockSpec((1, 512), lambda i: (0, 0)),
        ],
        out_shape=[jax.ShapeDtypeStruct((1, 512), jnp.float32)] * 2,
    )(enewf)


def _eupd_body(en_ref, ec_ref, sc_ref, sh_ref, o_ref, es_ref):
    i = pl.program_id(0)

    @pl.when(i == 0)
    def _():
        es_ref[...] = jnp.zeros_like(es_ref)

    out = ec_ref[...] + jnp.maximum(
        en_ref[...] * sc_ref[...] + sh_ref[...], 0.0
    )
    o_ref[...] = out
    es_ref[...] += jnp.sum(out, axis=0, keepdims=True)


def _e_update(enewf, ecf, scale, shift):
    return pl.pallas_call(
        _eupd_body,
        grid=(_SNB,),
        in_specs=[
            pl.BlockSpec((_SBLK, 512), lambda i: (i, 0)),
            pl.BlockSpec((_SBLK, 512), lambda i: (i, 0)),
            pl.BlockSpec((1, 512), lambda i: (0, 0)),
            pl.BlockSpec((1, 512), lambda i: (0, 0)),
        ],
        out_specs=[
            pl.BlockSpec((_SBLK, 512), lambda i: (i, 0)),
            pl.BlockSpec((1, 512), lambda i: (0, 0)),
        ],
        out_shape=[
            jax.ShapeDtypeStruct((_EROWS, 512), jnp.float32),
            jax.ShapeDtypeStruct((1, 512), jnp.float32),
        ],
    )(enewf, ecf, scale, shift)


def _ceupd_body(en_ref, ec_ref, sc_ref, sh_ref, w_ref, b_ref, oe_ref, oc_ref):
    out = ec_ref[...] + jnp.maximum(
        en_ref[...] * sc_ref[...] + sh_ref[...], 0.0
    )
    oe_ref[...] = out
    oc_ref[...] = (
        jnp.dot(out, w_ref[...], preferred_element_type=jnp.float32)
        + b_ref[...]
    )


def _ce_fused(enewf, ecf, scale, shift, w2, b2):
    return pl.pallas_call(
        _ceupd_body,
        grid=(_SNB,),
        in_specs=[
            pl.BlockSpec((_SBLK, 512), lambda i: (i, 0)),
            pl.BlockSpec((_SBLK, 512), lambda i: (i, 0)),
            pl.BlockSpec((1, 512), lambda i: (0, 0)),
            pl.BlockSpec((1, 512), lambda i: (0, 0)),
            pl.BlockSpec((512, 512), lambda i: (0, 0)),
            pl.BlockSpec((1, 512), lambda i: (0, 0)),
        ],
        out_specs=[
            pl.BlockSpec((_SBLK, 512), lambda i: (i, 0)),
            pl.BlockSpec((_SBLK, 512), lambda i: (i, 0)),
        ],
        out_shape=[
            jax.ShapeDtypeStruct((_EROWS, 512), jnp.float32),
            jax.ShapeDtypeStruct((_EROWS, 512), jnp.float32),
        ],
    )(enewf, ecf, scale, shift, w2, b2.reshape(1, 512))


def _proj_body(x_ref, w_ref, b_ref, ah_ref, sd_ref, ee_ref):
    res = (
        jnp.dot(x_ref[...], w_ref[...], preferred_element_type=jnp.float32)
        + b_ref[...]
    )
    ah_ref[...] = res[:, :HIDDEN]
    for c in range(NCHUNK):
        sd_ref[c] = jnp.concatenate(
            [
                res[:, 2 * HIDDEN + c * CW : 2 * HIDDEN + (c + 1) * CW],
                res[:, HIDDEN + c * CW : HIDDEN + (c + 1) * CW],
            ],
            axis=1,
        )
    for k2 in range(NCORES):
        ee_ref[k2] = res[:, 3 * HIDDEN + k2 * 128 : 3 * HIDDEN + (k2 + 1) * 128]


def _proj(h, w_cat, b_cat, bm=1000):
    m = h.shape[0]
    return pl.pallas_call(
        _proj_body,
        grid=(pl.cdiv(m, bm),),
        in_specs=[
            pl.BlockSpec((bm, HIDDEN), lambda i: (i, 0)),
            pl.BlockSpec((HIDDEN, 4 * HIDDEN), lambda i: (0, 0)),
            pl.BlockSpec((1, 4 * HIDDEN), lambda i: (0, 0)),
        ],
        out_specs=[
            pl.BlockSpec((bm, HIDDEN), lambda i: (i, 0)),
            pl.BlockSpec((NCHUNK, bm, 128), lambda i: (0, i, 0)),
            pl.BlockSpec((NCORES, bm, 128), lambda i: (0, i, 0)),
        ],
        out_shape=[
            jax.ShapeDtypeStruct((m, HIDDEN), jnp.float32),
            jax.ShapeDtypeStruct((NCHUNK, m, 128), jnp.float32),
            jax.ShapeDtypeStruct((NCORES, m, 128), jnp.float32),
        ],
    )(h, w_cat, b_cat.reshape(1, 4 * HIDDEN))


def _edge_body(
    gadj_hbm, dstp_hbm, ce_hbm, sd_hbm, ee_hbm,
    enew_hbm, acc_hbm,
    zbuf,
    gA, gB, dA, dB, ceA, ceB, sdA, sdB, eeA, eeB, sgA, sgB,
    acc_sh,
    iA, iB, dsA, dsB, gsA, gsB, geA, geB, csA, csB, wA, wB, ssA, ssB,
):
    core = lax.axis_index("c")
    sub = lax.axis_index("s")
    bufs = [
        (gA, dA, ceA, sdA, eeA, sgA, iA, dsA, gsA, geA, csA, wA, ssA),
        (gB, dB, ceB, sdB, eeB, sgB, iB, dsB, gsB, geB, csB, wB, ssB),
    ]

    zero16 = jnp.zeros((16,), jnp.float32)
    for i in range(ZROWS):
        for kk in range(8):
            zbuf[i, pl.ds(kk * 16, 16)] = zero16

    for p in range(NCHUNK // NCORES):
        c = core * (NCHUNK // NCORES) + p

        coff = pl.multiple_of(c * 128, 128)

        def ce_row(j):
            return pl.multiple_of(
                sub * (EDGES_PER_TILE // 2) + j * FROWS, 8
            )

        def fire_idx(j, P):
            g, dv = bufs[P][0], bufs[P][1]
            semI, semD = bufs[P][6], bufs[P][7]
            goff = pl.multiple_of(((c * NSUB + sub) * NBLKT + j) * 128, 8)
            pltpu.async_copy(gadj_hbm.at[pl.ds(goff, 128)], g, semI)
            doff = pl.multiple_of(sub * TPAD + j * BLK, 8)
            pltpu.async_copy(dstp_hbm.at[pl.ds(doff, BLK)], dv, semD)

        def fire_gather(j, P, wait_w, frows):
            g, dv, ceb, sdb, eeb, sgb = (
                bufs[P][0], bufs[P][1], bufs[P][2],
                bufs[P][3], bufs[P][4], bufs[P][5],
            )
            semI, semG, semE, semC, semW, semS = (
                bufs[P][6], bufs[P][8], bufs[P][9],
                bufs[P][10], bufs[P][11], bufs[P][12],
            )
            pltpu.make_async_copy(gadj_hbm.at[pl.ds(0, 128)], g, semI).wait()
            if wait_w:
                pltpu.make_async_copy(
                    ceb, enew_hbm.at[pl.ds(0, FROWS), pl.ds(0, 128)], semW
                ).wait()
                pltpu.make_async_copy(sgb, acc_sh.at[dv], semS).wait()
            pltpu.async_copy(sd_hbm.at[g.at[pl.ds(0, BLK)]], sdb, semG)
            pltpu.async_copy(ee_hbm.at[g.at[pl.ds(BLK, BLK)]], eeb, semE)
            ce_dst = ceb if frows == FROWS else ceb.at[pl.ds(0, frows)]
            pltpu.async_copy(
                ce_hbm.at[pl.ds(ce_row(j), frows), pl.ds(coff, 128)], ce_dst, semC
            )

        def compute_block(j, P, qrows, prefetch=None):
            g, dv, ceb, sdb, eeb, sgb = (
                bufs[P][0], bufs[P][1], bufs[P][2],
                bufs[P][3], bufs[P][4], bufs[P][5],
            )
            semD, semG, semE, semC, semW, semS = (
                bufs[P][7], bufs[P][8], bufs[P][9],
                bufs[P][10], bufs[P][11], bufs[P][12],
            )
            pltpu.make_async_copy(
                sd_hbm.at[g.at[pl.ds(0, BLK)]], sdb, semG
            ).wait()
            ce_dst = ceb if qrows == FROWS else ceb.at[pl.ds(0, qrows)]
            if prefetch is not None:
                prefetch()
            pltpu.make_async_copy(
                ee_hbm.at[g.at[pl.ds(BLK, BLK)]], eeb, semE
            ).wait()
            pltpu.make_async_copy(
                ce_hbm.at[pl.ds(0, qrows), pl.ds(0, 128)], ce_dst, semC
            ).wait()
            pltpu.make_async_copy(dstp_hbm.at[pl.ds(0, BLK)], dv, semD).wait()

            @plsc.parallel_loop(0, qrows, unroll=1)
            def comp(q):
                for half in range(2):
                    r = 2 * q + half
                    for kk in range(CW // 16):
                        csl = pl.ds(half * CW + kk * 16, 16)
                        sl = pl.ds(kk * 16, 16)
                        esl = pl.ds(p * CW + kk * 16, 16)
                        bsl = pl.ds(CW + kk * 16, 16)
                        en = ceb[q, csl] + sdb[r, sl] + eeb[r, esl]
                        ceb[q, csl] = en
                        sg = 1.0 / (1.0 + jnp.exp(-en))
                        sgb[r, bsl] = sg
                        sgb[r, sl] = sg * sdb[r, bsl]
            pltpu.async_copy(
                ce_dst,
                enew_hbm.at[pl.ds(ce_row(j), qrows), pl.ds(coff, 128)],
                semW,
            )
            pltpu.async_copy(sgb, acc_sh.at[dv], semS, add=True)

        for t in range(STRIPE // ZROWS):
            zr0 = pl.multiple_of(sub * STRIPE + t * ZROWS, 8)
            pltpu.sync_copy(zbuf, acc_sh.at[pl.ds(zr0, ZROWS)])
        plsc.subcore_barrier()

        def pair(j, carry):
            def pf_a():
                @pl.when(j + 2 < NBLKF)
                def _():
                    fire_idx(j + 2, 0)

            def pf_b():
                @pl.when(j + 3 < NBLKF)
                def _():
                    fire_idx(j + 3, 1)

            compute_block(j, 0, FROWS, prefetch=pf_a)

            @pl.when(j + 2 < NBLKF)
            def _():
                fire_gather(j + 2, 0, True, FROWS)

            compute_block(j + 1, 1, FROWS, prefetch=pf_b)

            @pl.when(j + 3 < NBLKF)
            def _():
                fire_gather(j + 3, 1, True, FROWS)

            return carry

        fire_idx(0, 0)
        fire_idx(1, 1)
        fire_gather(0, 0, False, FROWS)
        fire_gather(1, 1, False, FROWS)
        lax.fori_loop(0, NBLKF // 2, lambda jj, cc: pair(2 * jj, cc), 0)
        fire_idx(NBLKF, 0)
        fire_gather(NBLKF, 0, True, TROWS)
        compute_block(NBLKF, 0, TROWS)

        pltpu.make_async_copy(
            ceA.at[pl.ds(0, TROWS)],
            enew_hbm.at[pl.ds(0, TROWS), pl.ds(0, 128)],
            wA,
        ).wait()
        pltpu.make_async_copy(
            ceB, enew_hbm.at[pl.ds(0, FROWS), pl.ds(0, 128)], wB
        ).wait()
        pltpu.make_async_copy(sgA, acc_sh.at[dA], ssA).wait()
        pltpu.make_async_copy(sgB, acc_sh.at[dB], ssB).wait()
        plsc.subcore_barrier()

        fr0 = pl.multiple_of(sub * STRIPE, 8)
        fo0 = pl.multiple_of(c * N_ACC + sub * STRIPE, 8)
        pltpu.sync_copy(
            acc_sh.at[pl.ds(fr0, STRIPE)],
            acc_hbm.at[pl.ds(fo0, STRIPE)],
        )


_edge_kernel = functools.partial(
    pl.kernel,
    out_type=[
        jax.ShapeDtypeStruct((PROWS, 512), jnp.float32),
        jax.ShapeDtypeStruct((NCHUNK * N_ACC, 128), jnp.float32),
    ],
    mesh=plsc.VectorSubcoreMesh(core_axis_name="c", subcore_axis_name="s"),
    scratch_types=[
        pltpu.VMEM((ZROWS, 128), jnp.float32),
        pltpu.VMEM((128,), jnp.int32),
        pltpu.VMEM((128,), jnp.int32),
        pltpu.VMEM((BLK,), jnp.int32),
        pltpu.VMEM((BLK,), jnp.int32),
        pltpu.VMEM((FROWS, 128), jnp.float32),
        pltpu.VMEM((FROWS, 128), jnp.float32),
        pltpu.VMEM((BLK, 128), jnp.float32),
        pltpu.VMEM((BLK, 128), jnp.float32),
        pltpu.VMEM((BLK, 128), jnp.float32),
        pltpu.VMEM((BLK, 128), jnp.float32),
        pltpu.VMEM((BLK, 128), jnp.float32),
        pltpu.VMEM((BLK, 128), jnp.float32),
        pltpu.VMEM_SHARED((N_ACC, 128), jnp.float32),
    ] + [pltpu.SemaphoreType.DMA] * 14,
)(_edge_body)


def _bn(x, g, b, axis):
    mu = jnp.mean(x, axis=axis, keepdims=True)
    var = jnp.var(x, axis=axis, keepdims=True)
    return g * (x - mu) / jnp.sqrt(var + 1e-5) + b


def kernel(dataset_idx, edge_index, h, e, training_flag, params):
    src = edge_index[0]
    dst = edge_index[1]
    srcr = src.reshape(NSUB, EDGES_PER_TILE)
    dstr = dst.reshape(NSUB, EDGES_PER_TILE)
    pad = TPAD - EDGES_PER_TILE
    srcb = jnp.pad(srcr, ((0, 0), (0, pad))).reshape(NSUB, NBLKT, BLK)
    dstb = jnp.pad(dstr, ((0, 0), (0, pad))).reshape(NSUB, NBLKT, BLK)
    zpad = jnp.zeros((NSUB, NBLKT, 128 - 2 * BLK), jnp.int32)
    gadj = jnp.stack(
        [
            jnp.concatenate(
                [srcb + c * N_NODES, dstb + (c // 2) * N_NODES, zpad], axis=-1
            )
            for c in range(NCHUNK)
        ]
    ).reshape(-1)
    dstp = jnp.pad(
        dstr, ((0, 0), (0, pad)), constant_values=N_NODES
    ).reshape(-1)
    h = _matmul_bias(h, params["emb_h_W"], params["emb_h_b"], bm=1000)
    w2e, b2e = _pack2_weights(params["emb_e_W"], params["emb_e_b"])
    ecf = _matmul_bias(e.reshape(PROWS, 32), w2e, b2e, bm=2000)

    esum = None
    pend = None
    for lp in params["layers"]:
        w_cat = jnp.concatenate(
            [lp["A_W"], lp["B_W"], lp["D_W"], lp["E_W"]], axis=1
        )
        b_cat = jnp.concatenate(
            [lp["A_b"], lp["B_b"], lp["D_b"], lp["E_b"]], axis=0
        )
        Ah, sdT, eeT = _proj(h, w_cat, b_cat)
        w2c, b2c = _pack2_weights_ce(lp["C_W"], lp["C_b"])
        if pend is None:
            ceF = _matmul_bias(ecf, w2c, b2c, bm=2000)
        else:
            ecf, ceF = _ce_fused(pend[0], ecf, pend[1], pend[2], w2c, b2c)

        enewF, accF = _edge_kernel(
            gadj, dstp,
            ceF,
            sdT.reshape(NCHUNK * N_NODES, 128),
            eeT.reshape(NCORES * N_NODES, 128),
        )
        accR = accF.reshape(NCHUNK, N_ACC, 128)[:, :N_NODES]
        num = accR[:, :, :CW].transpose(1, 0, 2).reshape(N_NODES, HIDDEN)
        den = accR[:, :, CW:].transpose(1, 0, 2).reshape(N_NODES, HIDDEN)

        h_new = Ah + num / (den + 1e-6)
        h_new = jax.nn.relu(_bn(h_new, lp["bn_h_g"], lp["bn_h_b"], axis=0))
        h = h + h_new

        S, Q = _bn_stats(enewF)
        s4 = S.reshape(NCHUNK, 2, CW).sum(1)
        q4 = Q.reshape(NCHUNK, 2, CW).sum(1)
        mu = s4 / N_EDGES
        var = q4 / N_EDGES - mu * mu
        g4 = lp["bn_e_g"].reshape(NCHUNK, CW)
        b4 = lp["bn_e_b"].reshape(NCHUNK, CW)
        scale4 = g4 / jnp.sqrt(var + 1e-5)
        shift4 = b4 - mu * scale4
        scale = jnp.tile(scale4.reshape(NCHUNK, 1, CW), (1, 2, 1)).reshape(1, 512)
        shift = jnp.tile(shift4.reshape(NCHUNK, 1, CW), (1, 2, 1)).reshape(1, 512)
        pend = (enewF, scale, shift)

    ecf, esum = _e_update(pend[0], ecf, pend[1], pend[2])

    e_sum = esum.reshape(NCHUNK, 2, CW).sum(1).reshape(HIDDEN)
    hg = jnp.concatenate([jnp.sum(h, axis=0), e_sum])
    x = hg
    n = len(params["mlp_Ws"])
    for i in range(n):
        x = x @ params["mlp_Ws"][i] + params["mlp_bs"][i]
        if i < n - 1:
            x = jax.nn.relu(x)
    return x
